# Initial kernel scaffold; baseline (speedup 1.0000x reference)
import jax
import jax.numpy as jnp
from jax.experimental import pallas as pl


def kernel(inputs, embed_tables, W1, b1, W2, b2, Wf, bf, wide_w):
    raise NotImplementedError("write your pallas kernel here")


# speedup vs baseline: 7.3015x; 7.3015x over previous
import functools

import jax
import jax.numpy as jnp
from jax import lax
from jax.experimental import pallas as pl
from jax.experimental.pallas import tpu as pltpu
from jax.experimental.pallas import tpu_sc as plsc

_B = 16384
_F = 26
_V = 100000
_D = 16
_H = 256

_NC = 2
_NS = 16
_NW = _NC * _NS
_RPW = _B * _F // _NW
_GC = 128
_NGC = _RPW // _GC
_PERIOD = 13
_NVEC = _RPW // 16


def _sc_gather_body(idx_hbm, tab_hbm, ww_hbm, emb_hbm, wv_hbm,
                    idx_v, rows_v, wrow_v, widx_v, wout_v, off_v, sem, wsem):
    wid = lax.axis_index("s") * _NC + lax.axis_index("c")
    base = wid * _RPW

    pltpu.sync_copy(idx_hbm.at[pl.ds(wid * _NGC, _NGC)], idx_v)

    def mk_off(i, carry):
        for j in range(_GC // 16):
            t = i * _GC + j * 16 + lax.iota(jnp.int32, 16)
            off_v[i, pl.ds(j * 16, 16)] = (t % _F) * _V
        return carry

    lax.fori_loop(0, _PERIOD, mk_off, 0)

    def add_off(c, carry):
        p = c % _PERIOD
        for j in range(_GC // 16):
            s = j * 16
            idx_v[c, pl.ds(s, 16)] = (idx_v[c, pl.ds(s, 16)]
                                      + off_v[p, pl.ds(s, 16)])
        return carry

    lax.fori_loop(0, _NGC, add_off, 0)

    lanes = lax.iota(jnp.int32, 16)

    def gchunk(c, carry):
        ids = idx_v.at[c]
        cp = pltpu.async_copy(tab_hbm.at[ids], rows_v, sem)
        for j in range(_GC // 16):
            widx_v[0, pl.ds(j * 16, 16)] = lax.shift_right_logical(
                idx_v[c, pl.ds(j * 16, 16)], 4)
        wcp = pltpu.async_copy(ww_hbm.at[widx_v.at[0]], wrow_v, wsem)
        cp.wait()
        pltpu.sync_copy(rows_v, emb_hbm.at[pl.ds(base + c * _GC, _GC)])
        wcp.wait()
        for j in range(_GC // 16):
            sub = idx_v[c, pl.ds(j * 16, 16)] & 15
            wout_v[pl.ds(j * 16, 16)] = plsc.load_gather(
                wrow_v, [j * 16 + lanes, sub])
        pltpu.sync_copy(wout_v, wv_hbm.at[wid * _NGC + c])
        return carry

    lax.fori_loop(0, _NGC, gchunk, 0)


def _sc_gather(idx_flat, tab, ww):
    mesh = plsc.VectorSubcoreMesh(core_axis_name="c", subcore_axis_name="s")
    f = pl.kernel(
        _sc_gather_body,
        out_type=(
            jax.ShapeDtypeStruct((_B * _F, _D), jnp.float32),
            jax.ShapeDtypeStruct((_B * _F // _GC, _GC), jnp.float32),
        ),
        mesh=mesh,
        compiler_params=pltpu.CompilerParams(use_tc_tiling_on_sc=False,
                                             needs_layout_passes=False),
        scratch_types=[
            pltpu.VMEM((_NGC, _GC), jnp.int32),
            pltpu.VMEM((_GC, _D), jnp.float32),
            pltpu.VMEM((_GC, 16), jnp.float32),
            pltpu.VMEM((1, _GC), jnp.int32),
            pltpu.VMEM((_GC,), jnp.float32),
            pltpu.VMEM((_PERIOD, _GC), jnp.int32),
            pltpu.SemaphoreType.DMA,
            pltpu.SemaphoreType.DMA,
        ],
    )
    return f(idx_flat, tab, ww)


def _mlp_body(x_ref, wv_ref, w1_ref, b1_ref, w2_ref, b2_ref, wf_ref, bf_ref,
              o_ref):
    x = x_ref[...]
    h = jnp.maximum(
        jnp.dot(x, w1_ref[...], preferred_element_type=jnp.float32)
        + b1_ref[...], 0.0)
    h = jnp.maximum(
        jnp.dot(h, w2_ref[...], preferred_element_type=jnp.float32)
        + b2_ref[...], 0.0)
    deep = jnp.sum(h * wf_ref[...], axis=1, keepdims=True) + bf_ref[...]
    wide = jnp.sum(wv_ref[...], axis=1, keepdims=True)
    z = 0.5 * wide + 0.5 * deep
    o_ref[...] = 1.0 / (1.0 + jnp.exp(-z))


def _mlp(x, wv, w1, b1, w2, b2, wf, bf):
    blk = 2048
    grid = _B // blk
    return pl.pallas_call(
        _mlp_body,
        grid=(grid,),
        in_specs=[
            pl.BlockSpec((blk, _F * _D), lambda i: (i, 0)),
            pl.BlockSpec((blk, _F), lambda i: (i, 0)),
            pl.BlockSpec((_F * _D, _H), lambda i: (0, 0)),
            pl.BlockSpec((1, _H), lambda i: (0, 0)),
            pl.BlockSpec((_H, _H), lambda i: (0, 0)),
            pl.BlockSpec((1, _H), lambda i: (0, 0)),
            pl.BlockSpec((1, _H), lambda i: (0, 0)),
            pl.BlockSpec((1, 1), lambda i: (0, 0)),
        ],
        out_specs=pl.BlockSpec((blk, 1), lambda i: (i, 0)),
        out_shape=jax.ShapeDtypeStruct((_B, 1), jnp.float32),
    )(x, wv, w1, b1, w2, b2, wf, bf)


def kernel(inputs, embed_tables, W1, b1, W2, b2, Wf, bf, wide_w):
    idx_flat = inputs.astype(jnp.int32).reshape(_B * _F // _GC, _GC)
    tab = embed_tables.reshape(_F * _V, _D)
    emb_flat, wvals = _sc_gather(idx_flat, tab,
                                 wide_w.reshape(_F * _V // 16, 16))
    x = emb_flat.reshape(_B, _F * _D)
    wv = wvals.reshape(_B, _F)
    return _mlp(x, wv, W1, b1.reshape(1, _H), W2, b2.reshape(1, _H),
                Wf.reshape(1, _H), bf.reshape(1, 1))


# speedup vs baseline: 12.6133x; 1.7275x over previous
import functools

import jax
import jax.numpy as jnp
from jax import lax
from jax.experimental import pallas as pl
from jax.experimental.pallas import tpu as pltpu
from jax.experimental.pallas import tpu_sc as plsc

_B = 16384
_F = 26
_V = 100000
_D = 16
_H = 256

_NC = 2
_NS = 16
_NW = _NC * _NS
_RPW = _B * _F // _NW
_GC = 128
_NGC = _RPW // _GC
_PERIOD = 13
_NVEC = _RPW // 16


_VG = _V // 16


def _sc_gather_body(idx_hbm, tab_hbm, ww_hbm, emb_hbm, wv_hbm,
                    idx_v, i16_v, lan_v, wi_v, w3_v, ww3_v,
                    rows_o, wout_v, off_v, sem0, sem1, wsem0, wsem1):
    wid = lax.axis_index("s") * _NC + lax.axis_index("c")
    base = wid * _RPW
    lanes = lax.iota(jnp.int32, 16)
    sems = (sem0, sem1)
    wsems = (wsem0, wsem1)

    pltpu.sync_copy(idx_hbm.at[pl.ds(wid * _NGC, _NGC)], idx_v)

    def mk_off(i, carry):
        for j in range(_GC // 16):
            t = i * _GC + j * 16 + lanes
            off_v[i, pl.ds(j * 16, 16)] = (t % _F) * _V
        return carry

    lax.fori_loop(0, _PERIOD, mk_off, 0)

    def fire(c, p):
        po = c % _PERIOD
        for j in range(_GC // 16):
            s = pl.ds(j * 16, 16)
            raw = idx_v[c, s]
            off = off_v[po, s]
            i16_v[p, 0, s] = off + lax.shift_right_logical(raw, 4)
            lan_v[p, 0, s] = raw & 15
            wi_v[p, 0, s] = lax.shift_right_logical(off + raw, 4)
        for d in range(1, _D):
            for j in range(_GC // 16):
                s = pl.ds(j * 16, 16)
                i16_v[p, d, s] = i16_v[p, 0, s] + d * _VG
        for d in range(_D):
            pltpu.async_copy(tab_hbm.at[i16_v.at[p, d]], w3_v.at[p, d],
                             sems[p])
        pltpu.async_copy(ww_hbm.at[wi_v.at[p, 0]], ww3_v.at[p], wsems[p])

    def drain(c, p):
        for d in range(_D):
            pltpu.make_async_copy(tab_hbm.at[i16_v.at[p, d]], w3_v.at[p, d],
                                  sems[p]).wait()
        pltpu.make_async_copy(ww_hbm.at[wi_v.at[p, 0]], ww3_v.at[p],
                              wsems[p]).wait()

    def extract(c, p):
        for j in range(_GC // 16):
            s = pl.ds(j * 16, 16)
            kv = j * 16 + lanes
            ln = lan_v[p, 0, s]
            for d in range(_D):
                vals = plsc.load_gather(w3_v.at[p, d], [kv, ln])
                plsc.store_scatter(rows_o, [kv, jnp.full((16,), d, jnp.int32)],
                                   vals)
            wout_v[s] = plsc.load_gather(ww3_v.at[p], [kv, ln])
        pltpu.sync_copy(rows_o, emb_hbm.at[pl.ds(base + c * _GC, _GC)])
        pltpu.sync_copy(wout_v, wv_hbm.at[wid * _NGC + c])

    fire(0, 0)

    def pair(i, carry):
        c0 = 2 * i
        c1 = c0 + 1
        fire(c1, 1)
        drain(c0, 0)
        extract(c0, 0)

        @pl.when(i < _NGC // 2 - 1)
        def _():
            fire(c1 + 1, 0)

        drain(c1, 1)
        extract(c1, 1)
        return carry

    lax.fori_loop(0, _NGC // 2, pair, 0)


def _sc_gather(idx_flat, tab, ww):
    mesh = plsc.VectorSubcoreMesh(core_axis_name="c", subcore_axis_name="s")
    f = pl.kernel(
        _sc_gather_body,
        out_type=(
            jax.ShapeDtypeStruct((_B * _F, _D), jnp.float32),
            jax.ShapeDtypeStruct((_B * _F // _GC, _GC), jnp.float32),
        ),
        mesh=mesh,
        compiler_params=pltpu.CompilerParams(use_tc_tiling_on_sc=False,
                                             needs_layout_passes=False),
        scratch_types=[
            pltpu.VMEM((_NGC, _GC), jnp.int32),
            pltpu.VMEM((2, _D, _GC), jnp.int32),
            pltpu.VMEM((2, 1, _GC), jnp.int32),
            pltpu.VMEM((2, 1, _GC), jnp.int32),
            pltpu.VMEM((2, _D, _GC, 16), jnp.float32),
            pltpu.VMEM((2, _GC, 16), jnp.float32),
            pltpu.VMEM((_GC, _D), jnp.float32),
            pltpu.VMEM((_GC,), jnp.float32),
            pltpu.VMEM((_PERIOD, _GC), jnp.int32),
            pltpu.SemaphoreType.DMA,
            pltpu.SemaphoreType.DMA,
            pltpu.SemaphoreType.DMA,
            pltpu.SemaphoreType.DMA,
        ],
    )
    return f(idx_flat, tab, ww)


def _mlp_body(x_ref, wv_ref, w1_ref, b1_ref, w2_ref, b2_ref, wf_ref, bf_ref,
              o_ref):
    x = x_ref[...]
    h = jnp.maximum(
        jnp.dot(x, w1_ref[...], preferred_element_type=jnp.float32)
        + b1_ref[...], 0.0)
    h = jnp.maximum(
        jnp.dot(h, w2_ref[...], preferred_element_type=jnp.float32)
        + b2_ref[...], 0.0)
    deep = jnp.sum(h * wf_ref[...], axis=1, keepdims=True) + bf_ref[...]
    wide = jnp.sum(wv_ref[...], axis=1, keepdims=True)
    z = 0.5 * wide + 0.5 * deep
    o_ref[...] = 1.0 / (1.0 + jnp.exp(-z))


def _mlp(x, wv, w1, b1, w2, b2, wf, bf):
    blk = 2048
    grid = _B // blk
    return pl.pallas_call(
        _mlp_body,
        grid=(grid,),
        in_specs=[
            pl.BlockSpec((blk, _F * _D), lambda i: (i, 0)),
            pl.BlockSpec((blk, _F), lambda i: (i, 0)),
            pl.BlockSpec((_F * _D, _H), lambda i: (0, 0)),
            pl.BlockSpec((1, _H), lambda i: (0, 0)),
            pl.BlockSpec((_H, _H), lambda i: (0, 0)),
            pl.BlockSpec((1, _H), lambda i: (0, 0)),
            pl.BlockSpec((1, _H), lambda i: (0, 0)),
            pl.BlockSpec((1, 1), lambda i: (0, 0)),
        ],
        out_specs=pl.BlockSpec((blk, 1), lambda i: (i, 0)),
        out_shape=jax.ShapeDtypeStruct((_B, 1), jnp.float32),
    )(x, wv, w1, b1, w2, b2, wf, bf)


def kernel(inputs, embed_tables, W1, b1, W2, b2, Wf, bf, wide_w):
    idx_flat = inputs.astype(jnp.int32).reshape(_B * _F // _GC, _GC)
    tab = jnp.swapaxes(embed_tables, 1, 2).reshape(_F * _D * _V // 16, 16)
    emb_flat, wvals = _sc_gather(idx_flat, tab,
                                 wide_w.reshape(_F * _V // 16, 16))
    x = emb_flat.reshape(_B, _F * _D)
    wv = wvals.reshape(_B, _F)
    return _mlp(x, wv, W1, b1.reshape(1, _H), W2, b2.reshape(1, _H),
                Wf.reshape(1, _H), bf.reshape(1, 1))


# speedup vs baseline: 14.5122x; 1.1505x over previous
import functools

import jax
import jax.numpy as jnp
from jax import lax
from jax.experimental import pallas as pl
from jax.experimental.pallas import tpu as pltpu
from jax.experimental.pallas import tpu_sc as plsc

_B = 16384
_F = 26
_V = 100000
_D = 16
_H = 256

_NC = 2
_NS = 16
_NW = _NC * _NS
_RPW = _B * _F // _NW
_GC = 128
_NGC = _RPW // _GC
_PERIOD = 13
_NVEC = _RPW // 16


_VG = _V // 16


def _stage_idx_and_offsets(idx_hbm, idx_v, off_v, wid, lanes):
    pltpu.sync_copy(idx_hbm.at[pl.ds(wid * _NGC, _NGC)], idx_v)

    def mk_off(i, carry):
        for j in range(_GC // 16):
            t = i * _GC + j * 16 + lanes
            off_v[i, pl.ds(j * 16, 16)] = (t % _F) * _V
        return carry

    lax.fori_loop(0, _PERIOD, mk_off, 0)


def _sc_emb_body(idx_hbm, tab_hbm, emb_hbm,
                 idx_v, i16_v, lan_v, w3_v, rows_o, off_v, sem0, sem1):
    wid = lax.axis_index("s") * _NC + lax.axis_index("c")
    base = wid * _RPW
    lanes = lax.iota(jnp.int32, 16)
    sems = (sem0, sem1)
    _stage_idx_and_offsets(idx_hbm, idx_v, off_v, wid, lanes)

    def fire(c, p):
        po = c % _PERIOD
        for j in range(_GC // 16):
            s = pl.ds(j * 16, 16)
            raw = idx_v[c, s]
            off = off_v[po, s]
            i16_v[p, 0, s] = off + lax.shift_right_logical(raw, 4)
            lan_v[p, 0, s] = raw & 15
        for d in range(1, _D):
            for j in range(_GC // 16):
                s = pl.ds(j * 16, 16)
                i16_v[p, d, s] = i16_v[p, 0, s] + d * _VG
        for d in range(_D):
            pltpu.async_copy(tab_hbm.at[i16_v.at[p, d]], w3_v.at[p, d],
                             sems[p])

    def drain(c, p):
        for d in range(_D):
            pltpu.make_async_copy(tab_hbm.at[i16_v.at[p, d]], w3_v.at[p, d],
                                  sems[p]).wait()

    def extract(c, p):
        for j in range(_GC // 16):
            s = pl.ds(j * 16, 16)
            kv = j * 16 + lanes
            ln = lan_v[p, 0, s]
            for d in range(_D):
                vals = plsc.load_gather(w3_v.at[p, d], [kv, ln])
                plsc.store_scatter(rows_o, [kv, jnp.full((16,), d, jnp.int32)],
                                   vals)
        pltpu.sync_copy(rows_o, emb_hbm.at[pl.ds(base + c * _GC, _GC)])

    fire(0, 0)

    def pair(i, carry):
        c0 = 2 * i
        c1 = c0 + 1
        fire(c1, 1)
        drain(c0, 0)
        extract(c0, 0)

        @pl.when(i < _NGC // 2 - 1)
        def _():
            fire(c1 + 1, 0)

        drain(c1, 1)
        extract(c1, 1)
        return carry

    lax.fori_loop(0, _NGC // 2, pair, 0)


def _sc_wide_body(idx_hbm, ww_hbm, wv_hbm,
                  idx_v, wi_v, lan_v, ww3_v, wout_v, off_v, wsem0, wsem1):
    wid = lax.axis_index("s") * _NC + lax.axis_index("c")
    lanes = lax.iota(jnp.int32, 16)
    wsems = (wsem0, wsem1)
    _stage_idx_and_offsets(idx_hbm, idx_v, off_v, wid, lanes)

    def fire(c, p):
        po = c % _PERIOD
        for j in range(_GC // 16):
            s = pl.ds(j * 16, 16)
            raw = idx_v[c, s]
            g = off_v[po, s] + raw
            wi_v[p, 0, s] = lax.shift_right_logical(g, 4)
            lan_v[p, 0, s] = raw & 15
        pltpu.async_copy(ww_hbm.at[wi_v.at[p, 0]], ww3_v.at[p], wsems[p])

    def drain_extract(c, p):
        pltpu.make_async_copy(ww_hbm.at[wi_v.at[p, 0]], ww3_v.at[p],
                              wsems[p]).wait()
        for j in range(_GC // 16):
            s = pl.ds(j * 16, 16)
            wout_v[s] = plsc.load_gather(ww3_v.at[p],
                                         [j * 16 + lanes, lan_v[p, 0, s]])
        pltpu.sync_copy(wout_v, wv_hbm.at[wid * _NGC + c])

    fire(0, 0)

    def pair(i, carry):
        c0 = 2 * i
        c1 = c0 + 1
        fire(c1, 1)
        drain_extract(c0, 0)

        @pl.when(i < _NGC // 2 - 1)
        def _():
            fire(c1 + 1, 0)

        drain_extract(c1, 1)
        return carry

    lax.fori_loop(0, _NGC // 2, pair, 0)


_SC_PARAMS = pltpu.CompilerParams(use_tc_tiling_on_sc=False,
                                  needs_layout_passes=False)


def _sc_emb(idx_flat, tab):
    mesh = plsc.VectorSubcoreMesh(core_axis_name="c", subcore_axis_name="s")
    f = pl.kernel(
        _sc_emb_body,
        out_type=jax.ShapeDtypeStruct((_B * _F, _D), jnp.float32),
        mesh=mesh,
        compiler_params=_SC_PARAMS,
        scratch_types=[
            pltpu.VMEM((_NGC, _GC), jnp.int32),
            pltpu.VMEM((2, _D, _GC), jnp.int32),
            pltpu.VMEM((2, 1, _GC), jnp.int32),
            pltpu.VMEM((2, _D, _GC, 16), jnp.float32),
            pltpu.VMEM((_GC, _D), jnp.float32),
            pltpu.VMEM((_PERIOD, _GC), jnp.int32),
            pltpu.SemaphoreType.DMA,
            pltpu.SemaphoreType.DMA,
        ],
    )
    return f(idx_flat, tab)


def _sc_wide(idx_flat, ww):
    mesh = plsc.VectorSubcoreMesh(core_axis_name="c", subcore_axis_name="s")
    f = pl.kernel(
        _sc_wide_body,
        out_type=jax.ShapeDtypeStruct((_B * _F // _GC, _GC), jnp.float32),
        mesh=mesh,
        compiler_params=_SC_PARAMS,
        scratch_types=[
            pltpu.VMEM((_NGC, _GC), jnp.int32),
            pltpu.VMEM((2, 1, _GC), jnp.int32),
            pltpu.VMEM((2, 1, _GC), jnp.int32),
            pltpu.VMEM((2, _GC, 16), jnp.float32),
            pltpu.VMEM((_GC,), jnp.float32),
            pltpu.VMEM((_PERIOD, _GC), jnp.int32),
            pltpu.SemaphoreType.DMA,
            pltpu.SemaphoreType.DMA,
        ],
    )
    return f(idx_flat, ww)


def _mlp_body(x_ref, wv_ref, w1_ref, b1_ref, w2_ref, b2_ref, wf_ref, bf_ref,
              o_ref):
    x = x_ref[...]
    h = jnp.maximum(
        jnp.dot(x, w1_ref[...], preferred_element_type=jnp.float32)
        + b1_ref[...], 0.0)
    h = jnp.maximum(
        jnp.dot(h, w2_ref[...], preferred_element_type=jnp.float32)
        + b2_ref[...], 0.0)
    deep = jnp.sum(h * wf_ref[...], axis=1, keepdims=True) + bf_ref[...]
    wide = jnp.sum(wv_ref[...], axis=1, keepdims=True)
    z = 0.5 * wide + 0.5 * deep
    o_ref[...] = 1.0 / (1.0 + jnp.exp(-z))


def _mlp(x, wv, w1, b1, w2, b2, wf, bf):
    blk = 2048
    grid = _B // blk
    return pl.pallas_call(
        _mlp_body,
        grid=(grid,),
        in_specs=[
            pl.BlockSpec((blk, _F * _D), lambda i: (i, 0)),
            pl.BlockSpec((blk, _F), lambda i: (i, 0)),
            pl.BlockSpec((_F * _D, _H), lambda i: (0, 0)),
            pl.BlockSpec((1, _H), lambda i: (0, 0)),
            pl.BlockSpec((_H, _H), lambda i: (0, 0)),
            pl.BlockSpec((1, _H), lambda i: (0, 0)),
            pl.BlockSpec((1, _H), lambda i: (0, 0)),
            pl.BlockSpec((1, 1), lambda i: (0, 0)),
        ],
        out_specs=pl.BlockSpec((blk, 1), lambda i: (i, 0)),
        out_shape=jax.ShapeDtypeStruct((_B, 1), jnp.float32),
    )(x, wv, w1, b1, w2, b2, wf, bf)


def kernel(inputs, embed_tables, W1, b1, W2, b2, Wf, bf, wide_w):
    idx_flat = inputs.astype(jnp.int32).reshape(_B * _F // _GC, _GC)
    tab = jnp.swapaxes(embed_tables, 1, 2).reshape(_F * _D * _V // 16, 16)
    emb_flat = _sc_emb(idx_flat, tab)
    wvals = _sc_wide(idx_flat, wide_w.reshape(_F * _V // 16, 16))
    x = emb_flat.reshape(_B, _F * _D)
    wv = wvals.reshape(_B, _F)
    return _mlp(x, wv, W1, b1.reshape(1, _H), W2, b2.reshape(1, _H),
                Wf.reshape(1, _H), bf.reshape(1, 1))
